# Initial kernel scaffold; baseline (speedup 1.0000x reference)
import jax
import jax.numpy as jnp
from jax.experimental import pallas as pl


def kernel(continuous_fields, tile_type_field, embed_table):
    raise NotImplementedError("write your pallas kernel here")


# speedup vs baseline: 7.1724x; 7.1724x over previous
import jax
import jax.numpy as jnp
from jax import lax
from jax.experimental import pallas as pl
from jax.experimental.pallas import tpu as pltpu
from jax.experimental.pallas import tpu_sc as plsc

B = 16384
L = 100
F = 64
NQ = L // 4
QW = 4 * F
NC, NS = 2, 16
NW = NC * NS
RPW = B // NW
CH = 8
NCHUNK = RPW // CH


def _body(cont_hbm, idx_hbm, qtable_hbm, out_hbm, raw_v, cbuf_v, buf_v, sem_in, sem_out, *pid_refs):
    wid = lax.axis_index("s") * NC + lax.axis_index("c")
    row0 = wid * RPW
    lane = jax.lax.iota(jnp.int32, 16)

    def chunk_body(g, carry):
        base = row0 + g * CH
        pltpu.sync_copy(idx_hbm.at[pl.ds(base, CH)], raw_v)
        for i in range(CH):
            pidr = pid_refs[i]
            rvec = jnp.full((16,), i, dtype=jnp.int32)
            for c0 in (0, 16):
                col = jnp.minimum((c0 + lane) * 4, L - 4)
                t0 = plsc.load_gather(raw_v, [rvec, col])
                t1 = plsc.load_gather(raw_v, [rvec, col + 1])
                t2 = plsc.load_gather(raw_v, [rvec, col + 2])
                t3 = plsc.load_gather(raw_v, [rvec, col + 3])
                pidr[pl.ds(c0, 16)] = ((t0 * 4 + t1) * 4 + t2) * 4 + t3
        descs = [pltpu.async_copy(cont_hbm.at[pl.ds(base, CH)], cbuf_v, sem_in)]
        for i in range(CH):
            descs.append(
                pltpu.async_copy(
                    qtable_hbm.at[pid_refs[i]],
                    buf_v.at[i, pl.ds(1, 32)],
                    sem_in,
                )
            )
        for d in descs:
            d.wait()
        for i in range(CH):
            for j in range(QW // 16):
                buf_v[i, 0, pl.ds(16 * j, 16)] = cbuf_v[i, pl.ds(16 * j, 16)]
        out_descs = []
        for i in range(CH):
            out_descs.append(
                pltpu.async_copy(buf_v.at[i, pl.ds(0, NQ + 1)], out_hbm.at[base + i], sem_out)
            )
        for d in out_descs:
            d.wait()
        return carry

    lax.fori_loop(0, NCHUNK, chunk_body, 0)


def kernel(continuous_fields, tile_type_field, embed_table):
    idx = tile_type_field.astype(jnp.int32)
    d = jnp.arange(256, dtype=jnp.int32)
    digits = [(d // 64) % 4, (d // 16) % 4, (d // 4) % 4, d % 4]
    qtable = jnp.concatenate(
        [embed_table[jnp.clip(t + 1, 0, 3)] for t in digits], axis=1
    )
    mesh = plsc.VectorSubcoreMesh(core_axis_name="c", subcore_axis_name="s")
    out3 = pl.kernel(
        _body,
        out_type=jax.ShapeDtypeStruct((B, NQ + 1, QW), jnp.float32),
        mesh=mesh,
        compiler_params=pltpu.CompilerParams(
            use_tc_tiling_on_sc=False, needs_layout_passes=False
        ),
        scratch_types=[
            pltpu.VMEM((CH, L), jnp.int32),
            pltpu.VMEM((CH, QW), jnp.float32),
            pltpu.VMEM((CH, 33, QW), jnp.float32),
            pltpu.SemaphoreType.DMA,
            pltpu.SemaphoreType.DMA,
        ]
        + [pltpu.VMEM((32,), jnp.int32) for _ in range(CH)],
    )(continuous_fields, idx, qtable)
    return out3.reshape(B, (NQ + 1) * QW)
